# baseline (speedup 1.0000x reference)
import jax
import jax.numpy as jnp
from jax.experimental import pallas as pl

R = 4
C_OUT = 2
BLOCK_E = 128000


def _body(W_ref, w_ref, out_ref, filt_ref):
    Wv = W_ref[...]
    m = jnp.max(Wv, axis=1, keepdims=True)
    e = jnp.exp(Wv - m)
    f = e / jnp.sum(e, axis=1, keepdims=True)
    filt_ref[...] = f
    wb = w_ref[...]
    out_ref[...] = jax.lax.dot_general(
        f, wb, (((1,), (0,)), ((), ())), preferred_element_type=jnp.float32
    )


def kernel(w, edge_index, W):
    del edge_index
    E = w.shape[1]
    grid = E // BLOCK_E
    w_sum, Filter = pl.pallas_call(
        _body,
        grid=(grid,),
        in_specs=[
            pl.BlockSpec((C_OUT, R), lambda i: (0, 0)),
            pl.BlockSpec((R, BLOCK_E), lambda i: (0, i)),
        ],
        out_specs=[
            pl.BlockSpec((C_OUT, BLOCK_E), lambda i: (0, i)),
            pl.BlockSpec((C_OUT, R), lambda i: (0, 0)),
        ],
        out_shape=[
            jax.ShapeDtypeStruct((C_OUT, E), jnp.float32),
            jax.ShapeDtypeStruct((C_OUT, R), jnp.float32),
        ],
    )(W, w)
    return (w_sum, Filter)


# speedup vs baseline: 1.3001x; 1.3001x over previous
import jax
import jax.numpy as jnp
from jax.experimental import pallas as pl

R = 4
C_OUT = 2
BLOCK_E = 256000


def _body(W_ref, w_ref, out_ref, filt_ref):
    Wv = W_ref[...]
    m = jnp.max(Wv, axis=1, keepdims=True)
    e = jnp.exp(Wv - m)
    f = e / jnp.sum(e, axis=1, keepdims=True)
    filt_ref[...] = f
    wb = w_ref[...]
    out_ref[...] = jax.lax.dot_general(
        f, wb, (((1,), (0,)), ((), ())), preferred_element_type=jnp.float32
    )


def kernel(w, edge_index, W):
    del edge_index
    E = w.shape[1]
    grid = E // BLOCK_E
    w_sum, Filter = pl.pallas_call(
        _body,
        grid=(grid,),
        in_specs=[
            pl.BlockSpec((C_OUT, R), lambda i: (0, 0)),
            pl.BlockSpec((R, BLOCK_E), lambda i: (0, i)),
        ],
        out_specs=[
            pl.BlockSpec((C_OUT, BLOCK_E), lambda i: (0, i)),
            pl.BlockSpec((C_OUT, R), lambda i: (0, 0)),
        ],
        out_shape=[
            jax.ShapeDtypeStruct((C_OUT, E), jnp.float32),
            jax.ShapeDtypeStruct((C_OUT, R), jnp.float32),
        ],
    )(W, w)
    return (w_sum, Filter)


# speedup vs baseline: 1.4211x; 1.0930x over previous
import jax
import jax.numpy as jnp
from jax.experimental import pallas as pl

R = 4
C_OUT = 2
BLOCK_E = 640000


def _body(W_ref, w_ref, out_ref, filt_ref):
    Wv = W_ref[...]
    m = jnp.max(Wv, axis=1, keepdims=True)
    e = jnp.exp(Wv - m)
    f = e / jnp.sum(e, axis=1, keepdims=True)
    filt_ref[...] = f
    wb = w_ref[...]
    out_ref[...] = jax.lax.dot_general(
        f, wb, (((1,), (0,)), ((), ())), preferred_element_type=jnp.float32
    )


def kernel(w, edge_index, W):
    del edge_index
    E = w.shape[1]
    grid = E // BLOCK_E
    w_sum, Filter = pl.pallas_call(
        _body,
        grid=(grid,),
        in_specs=[
            pl.BlockSpec((C_OUT, R), lambda i: (0, 0)),
            pl.BlockSpec((R, BLOCK_E), lambda i: (0, i)),
        ],
        out_specs=[
            pl.BlockSpec((C_OUT, BLOCK_E), lambda i: (0, i)),
            pl.BlockSpec((C_OUT, R), lambda i: (0, 0)),
        ],
        out_shape=[
            jax.ShapeDtypeStruct((C_OUT, E), jnp.float32),
            jax.ShapeDtypeStruct((C_OUT, R), jnp.float32),
        ],
    )(W, w)
    return (w_sum, Filter)


# speedup vs baseline: 1.4587x; 1.0264x over previous
import jax
import jax.numpy as jnp
from jax.experimental import pallas as pl

R = 4
C_OUT = 2
BLOCK_E = 800000


def _body(W_ref, w_ref, out_ref, filt_ref):
    Wv = W_ref[...]
    m = jnp.max(Wv, axis=1, keepdims=True)
    e = jnp.exp(Wv - m)
    f = e / jnp.sum(e, axis=1, keepdims=True)
    filt_ref[...] = f
    wb = w_ref[...]
    out_ref[...] = jax.lax.dot_general(
        f, wb, (((1,), (0,)), ((), ())), preferred_element_type=jnp.float32
    )


def kernel(w, edge_index, W):
    del edge_index
    E = w.shape[1]
    grid = E // BLOCK_E
    w_sum, Filter = pl.pallas_call(
        _body,
        grid=(grid,),
        in_specs=[
            pl.BlockSpec((C_OUT, R), lambda i: (0, 0)),
            pl.BlockSpec((R, BLOCK_E), lambda i: (0, i)),
        ],
        out_specs=[
            pl.BlockSpec((C_OUT, BLOCK_E), lambda i: (0, i)),
            pl.BlockSpec((C_OUT, R), lambda i: (0, 0)),
        ],
        out_shape=[
            jax.ShapeDtypeStruct((C_OUT, E), jnp.float32),
            jax.ShapeDtypeStruct((C_OUT, R), jnp.float32),
        ],
    )(W, w)
    return (w_sum, Filter)
